# baseline (speedup 1.0000x reference)
import functools

import jax
import jax.numpy as jnp
from jax import lax
from jax.experimental import pallas as pl
from jax.experimental.pallas import tpu as pltpu
from jax.experimental.pallas import tpu_sc as plsc

N_NODES = 10000
N_EDGES = 320000
FEATS = 128
NC, NS = 2, 16
NPAD = 10240
SLICE = NPAD // NS
CHUNK = 80
EPW = N_EDGES // (NC * NS)
NCHUNK = EPW // CHUNK
G = 25
NG = NCHUNK // G
NR = 3
RB = 2000
NRB = N_NODES // RB

_mesh = plsc.VectorSubcoreMesh(core_axis_name="c", subcore_axis_name="s")


@functools.partial(
    pl.kernel,
    out_type=jax.ShapeDtypeStruct((NC, NPAD), jnp.float32),
    mesh=_mesh,
    scratch_types=[
        pltpu.VMEM((2, G, CHUNK), jnp.int32),
        pltpu.VMEM((CHUNK,), jnp.float32),
        pltpu.VMEM((SLICE,), jnp.float32),
        pltpu.VMEM_SHARED((NPAD,), jnp.float32),
        pltpu.SemaphoreType.DMA((2,)),
        pltpu.SemaphoreType.DMA((2,)),
    ],
)
def _deg_kernel(dst5_hbm, degp_hbm, idx_v, ones_v, zer_v, deg_sh, isems, csems):
    c = lax.axis_index("c")
    s = lax.axis_index("s")
    one = jnp.full((16,), 1.0, jnp.float32)
    for k in range(CHUNK // 16):
        ones_v[pl.ds(k * 16, 16)] = one
    zero = jnp.zeros((16,), jnp.float32)
    for k in range(SLICE // 16):
        zer_v[pl.ds(k * 16, 16)] = zero
    pltpu.sync_copy(zer_v, deg_sh.at[pl.ds(s * SLICE, SLICE)])
    plsc.subcore_barrier()
    pltpu.sync_copy(dst5_hbm.at[c, s, 0], idx_v.at[0])

    def step(j, carry):
        g = lax.div(j, G)
        jj = lax.rem(j, G)
        gb = lax.rem(g, 2)
        ngb = 1 - gb
        b = lax.rem(j, 2)

        @pl.when((jj == 0) & (g + 1 < NG))
        def _():
            pltpu.async_copy(dst5_hbm.at[c, s, g + 1], idx_v.at[ngb],
                             isems.at[ngb])

        @pl.when((jj == G - 1) & (g + 1 < NG))
        def _():
            pltpu.make_async_copy(dst5_hbm.at[c, s, g + 1], idx_v.at[ngb],
                                  isems.at[ngb]).wait()

        @pl.when(j >= 2)
        def _():
            pltpu.make_async_copy(ones_v, deg_sh.at[idx_v.at[gb, jj]],
                                  csems.at[b]).wait()

        pltpu.async_copy(ones_v, deg_sh.at[idx_v.at[gb, jj]], csems.at[b],
                         add=True)
        return carry

    lax.fori_loop(0, NCHUNK, step, 0)
    for p in range(2):
        pltpu.make_async_copy(ones_v, deg_sh.at[idx_v.at[0, 0]],
                              csems.at[(NCHUNK - 1 - p) % 2]).wait()
    plsc.subcore_barrier()
    pltpu.sync_copy(deg_sh.at[pl.ds(s * SLICE, SLICE)],
                    degp_hbm.at[c, pl.ds(s * SLICE, SLICE)])


@functools.partial(
    pl.kernel,
    out_type=jax.ShapeDtypeStruct((NC, NPAD, FEATS), jnp.float32),
    mesh=_mesh,
    scratch_types=[
        pltpu.VMEM((2, G, CHUNK), jnp.int32),
        pltpu.VMEM((2, G, CHUNK), jnp.int32),
        pltpu.VMEM((NR, CHUNK, FEATS), jnp.float32),
        pltpu.VMEM_SHARED((NPAD, FEATS), jnp.float32),
        pltpu.SemaphoreType.DMA((NR,)),
        pltpu.SemaphoreType.DMA((2,)),
        pltpu.SemaphoreType.DMA((2,)),
        pltpu.SemaphoreType.DMA((NR,)),
    ],
)
def _agg_kernel(h_hbm, ei_hbm, aggp_hbm, src_v, dst_v, rows_v,
                agg_sh, sems, ssems, dsems, scsems):
    c = lax.axis_index("c")
    s = lax.axis_index("s")
    zero = jnp.zeros((16,), jnp.float32)

    def zrow(i, carry):
        for k in range(FEATS // 16):
            rows_v[0, i, pl.ds(k * 16, 16)] = zero
        return carry

    lax.fori_loop(0, CHUNK, zrow, 0)
    for b in range(SLICE // CHUNK):
        pltpu.sync_copy(rows_v.at[0],
                        agg_sh.at[pl.ds(s * SLICE + b * CHUNK, CHUNK)])
    plsc.subcore_barrier()

    pltpu.sync_copy(ei_hbm.at[0, c, s, 0], src_v.at[0])
    pltpu.sync_copy(ei_hbm.at[1, c, s, 0], dst_v.at[0])
    pltpu.async_copy(h_hbm.at[src_v.at[0, 0]], rows_v.at[0], sems.at[0])

    def step(j, carry):
        g = lax.div(j, G)
        jj = lax.rem(j, G)
        gb = lax.rem(g, 2)
        ngb = 1 - gb
        b = lax.rem(j, NR)
        nb = lax.rem(j + 1, NR)

        @pl.when((jj == 0) & (g + 1 < NG))
        def _():
            pltpu.async_copy(ei_hbm.at[0, c, s, g + 1],
                             src_v.at[ngb], ssems.at[ngb])
            pltpu.async_copy(ei_hbm.at[1, c, s, g + 1],
                             dst_v.at[ngb], dsems.at[ngb])

        @pl.when((jj == G - 1) & (g + 1 < NG))
        def _():
            pltpu.make_async_copy(ei_hbm.at[0, c, s, g + 1],
                                  src_v.at[ngb], ssems.at[ngb]).wait()
            pltpu.make_async_copy(ei_hbm.at[1, c, s, g + 1],
                                  dst_v.at[ngb], dsems.at[ngb]).wait()

        @pl.when(j >= NR - 1)
        def _():
            pltpu.make_async_copy(rows_v.at[nb],
                                  agg_sh.at[dst_v.at[gb, jj]],
                                  scsems.at[nb]).wait()

        @pl.when(j + 1 < NCHUNK)
        def _():
            g1 = lax.div(j + 1, G)
            jj1 = lax.rem(j + 1, G)
            pltpu.async_copy(h_hbm.at[src_v.at[lax.rem(g1, 2), jj1]],
                             rows_v.at[nb], sems.at[nb])

        pltpu.make_async_copy(h_hbm.at[src_v.at[gb, jj]], rows_v.at[b],
                              sems.at[b]).wait()
        pltpu.async_copy(rows_v.at[b], agg_sh.at[dst_v.at[gb, jj]],
                         scsems.at[b], add=True)
        return carry

    lax.fori_loop(0, NCHUNK, step, 0)
    for p in range(NR - 1):
        pltpu.make_async_copy(rows_v.at[0], agg_sh.at[dst_v.at[0, 0]],
                              scsems.at[(NCHUNK - 1 - p) % NR]).wait()
    plsc.subcore_barrier()
    pltpu.sync_copy(agg_sh.at[pl.ds(s * SLICE, SLICE)],
                    aggp_hbm.at[c, pl.ds(s * SLICE, SLICE)])


def _h_body(degp_ref, feat_ref, h_ref, norm_ref):
    deg = degp_ref[0] + degp_ref[1]
    norm = lax.rsqrt(jnp.maximum(deg, 1.0))[:N_NODES, None]
    norm_ref[...] = norm
    h_ref[...] = feat_ref[...] * norm


_h_call = pl.pallas_call(
    _h_body,
    out_shape=(jax.ShapeDtypeStruct((N_NODES, FEATS), jnp.float32),
               jax.ShapeDtypeStruct((N_NODES, 1), jnp.float32)),
)


def _out_body(aggp_ref, w_ref, norm_ref, bias_ref, out_ref):
    agg = aggp_ref[0] + aggp_ref[1]
    r = jnp.dot(agg, w_ref[...], preferred_element_type=jnp.float32)
    out_ref[...] = r * norm_ref[...] + bias_ref[...]


_out_call = pl.pallas_call(
    _out_body,
    grid=(NRB,),
    in_specs=[
        pl.BlockSpec((NC, RB, FEATS), lambda i: (0, i, 0)),
        pl.BlockSpec((FEATS, FEATS), lambda i: (0, 0)),
        pl.BlockSpec((RB, 1), lambda i: (i, 0)),
        pl.BlockSpec((1, FEATS), lambda i: (0, 0)),
    ],
    out_specs=pl.BlockSpec((RB, FEATS), lambda i: (i, 0)),
    out_shape=jax.ShapeDtypeStruct((N_NODES, FEATS), jnp.float32),
)


def kernel(feat, edge_index, weight, bias):
    ei = edge_index.reshape(2, NC, NS, NG, G, CHUNK)
    degp = _deg_kernel(edge_index[1].reshape(NC, NS, NG, G, CHUNK))
    h, norm = _h_call(degp, feat)
    aggp = _agg_kernel(h, ei)
    return _out_call(aggp, weight, norm, bias.reshape(1, FEATS))


# speedup vs baseline: 1.0706x; 1.0706x over previous
import functools

import jax
import jax.numpy as jnp
from jax import lax
from jax.experimental import pallas as pl
from jax.experimental.pallas import tpu as pltpu
from jax.experimental.pallas import tpu_sc as plsc

N_NODES = 10000
N_EDGES = 320000
FEATS = 128
NC, NS = 2, 16
NPAD = 10240
SLICE = NPAD // NS
CHUNK = 80
EPW = N_EDGES // (NC * NS)
NCHUNK = EPW // CHUNK
G = 25
NG = NCHUNK // G
NR = 3
RB = 2000
NRB = N_NODES // RB

_mesh = plsc.VectorSubcoreMesh(core_axis_name="c", subcore_axis_name="s")


@functools.partial(
    pl.kernel,
    out_type=jax.ShapeDtypeStruct((NC, NPAD), jnp.float32),
    mesh=_mesh,
    scratch_types=[
        pltpu.VMEM((2, G, CHUNK), jnp.int32),
        pltpu.VMEM((CHUNK,), jnp.float32),
        pltpu.VMEM((SLICE,), jnp.float32),
        pltpu.VMEM_SHARED((NPAD,), jnp.float32),
        pltpu.SemaphoreType.DMA((2,)),
        pltpu.SemaphoreType.DMA((2,)),
    ],
)
def _deg_kernel(ei_hbm, degp_hbm, idx_v, ones_v, zer_v, deg_sh, isems, csems):
    c = lax.axis_index("c")
    s = lax.axis_index("s")
    one = jnp.full((16,), 1.0, jnp.float32)
    for k in range(CHUNK // 16):
        ones_v[pl.ds(k * 16, 16)] = one
    zero = jnp.zeros((16,), jnp.float32)
    for k in range(SLICE // 16):
        zer_v[pl.ds(k * 16, 16)] = zero
    pltpu.sync_copy(zer_v, deg_sh.at[pl.ds(s * SLICE, SLICE)])
    plsc.subcore_barrier()
    pltpu.sync_copy(ei_hbm.at[1, c, s, 0], idx_v.at[0])

    def step(j, carry):
        g = lax.div(j, G)
        jj = lax.rem(j, G)
        gb = lax.rem(g, 2)
        ngb = 1 - gb
        b = lax.rem(j, 2)

        @pl.when((jj == 0) & (g + 1 < NG))
        def _():
            pltpu.async_copy(ei_hbm.at[1, c, s, g + 1], idx_v.at[ngb],
                             isems.at[ngb])

        @pl.when((jj == G - 1) & (g + 1 < NG))
        def _():
            pltpu.make_async_copy(ei_hbm.at[1, c, s, g + 1], idx_v.at[ngb],
                                  isems.at[ngb]).wait()

        @pl.when(j >= 2)
        def _():
            pltpu.make_async_copy(ones_v, deg_sh.at[idx_v.at[gb, jj]],
                                  csems.at[b]).wait()

        pltpu.async_copy(ones_v, deg_sh.at[idx_v.at[gb, jj]], csems.at[b],
                         add=True)
        return carry

    lax.fori_loop(0, NCHUNK, step, 0)
    for p in range(2):
        pltpu.make_async_copy(ones_v, deg_sh.at[idx_v.at[0, 0]],
                              csems.at[(NCHUNK - 1 - p) % 2]).wait()
    plsc.subcore_barrier()
    pltpu.sync_copy(deg_sh.at[pl.ds(s * SLICE, SLICE)],
                    degp_hbm.at[c, pl.ds(s * SLICE, SLICE)])


@functools.partial(
    pl.kernel,
    out_type=jax.ShapeDtypeStruct((NC, NPAD, FEATS), jnp.float32),
    mesh=_mesh,
    scratch_types=[
        pltpu.VMEM((2, G, CHUNK), jnp.int32),
        pltpu.VMEM((2, G, CHUNK), jnp.int32),
        pltpu.VMEM((NR, CHUNK, FEATS), jnp.float32),
        pltpu.VMEM_SHARED((NPAD, FEATS), jnp.float32),
        pltpu.SemaphoreType.DMA((NR,)),
        pltpu.SemaphoreType.DMA((2,)),
        pltpu.SemaphoreType.DMA((2,)),
        pltpu.SemaphoreType.DMA((NR,)),
    ],
)
def _agg_kernel(h_hbm, ei_hbm, aggp_hbm, src_v, dst_v, rows_v,
                agg_sh, sems, ssems, dsems, scsems):
    c = lax.axis_index("c")
    s = lax.axis_index("s")
    zero = jnp.zeros((16,), jnp.float32)

    def zrow(i, carry):
        for k in range(FEATS // 16):
            rows_v[0, i, pl.ds(k * 16, 16)] = zero
        return carry

    lax.fori_loop(0, CHUNK, zrow, 0)
    for b in range(SLICE // CHUNK):
        pltpu.sync_copy(rows_v.at[0],
                        agg_sh.at[pl.ds(s * SLICE + b * CHUNK, CHUNK)])
    plsc.subcore_barrier()

    pltpu.sync_copy(ei_hbm.at[0, c, s, 0], src_v.at[0])
    pltpu.sync_copy(ei_hbm.at[1, c, s, 0], dst_v.at[0])
    pltpu.async_copy(h_hbm.at[src_v.at[0, 0]], rows_v.at[0], sems.at[0])

    def step(j, carry):
        g = lax.div(j, G)
        jj = lax.rem(j, G)
        gb = lax.rem(g, 2)
        ngb = 1 - gb
        b = lax.rem(j, NR)
        nb = lax.rem(j + 1, NR)

        @pl.when((jj == 0) & (g + 1 < NG))
        def _():
            pltpu.async_copy(ei_hbm.at[0, c, s, g + 1],
                             src_v.at[ngb], ssems.at[ngb])
            pltpu.async_copy(ei_hbm.at[1, c, s, g + 1],
                             dst_v.at[ngb], dsems.at[ngb])

        @pl.when((jj == G - 1) & (g + 1 < NG))
        def _():
            pltpu.make_async_copy(ei_hbm.at[0, c, s, g + 1],
                                  src_v.at[ngb], ssems.at[ngb]).wait()
            pltpu.make_async_copy(ei_hbm.at[1, c, s, g + 1],
                                  dst_v.at[ngb], dsems.at[ngb]).wait()

        @pl.when(j >= NR - 1)
        def _():
            pltpu.make_async_copy(rows_v.at[nb],
                                  agg_sh.at[dst_v.at[gb, jj]],
                                  scsems.at[nb]).wait()

        @pl.when(j + 1 < NCHUNK)
        def _():
            g1 = lax.div(j + 1, G)
            jj1 = lax.rem(j + 1, G)
            pltpu.async_copy(h_hbm.at[src_v.at[lax.rem(g1, 2), jj1]],
                             rows_v.at[nb], sems.at[nb])

        pltpu.make_async_copy(h_hbm.at[src_v.at[gb, jj]], rows_v.at[b],
                              sems.at[b]).wait()
        pltpu.async_copy(rows_v.at[b], agg_sh.at[dst_v.at[gb, jj]],
                         scsems.at[b], add=True)
        return carry

    lax.fori_loop(0, NCHUNK, step, 0)
    for p in range(NR - 1):
        pltpu.make_async_copy(rows_v.at[0], agg_sh.at[dst_v.at[0, 0]],
                              scsems.at[(NCHUNK - 1 - p) % NR]).wait()
    plsc.subcore_barrier()
    pltpu.sync_copy(agg_sh.at[pl.ds(s * SLICE, SLICE)],
                    aggp_hbm.at[c, pl.ds(s * SLICE, SLICE)])


def _h_body(degp_ref, feat_ref, h_ref, norm_ref):
    deg = degp_ref[0] + degp_ref[1]
    norm = lax.rsqrt(jnp.maximum(deg, 1.0))[:N_NODES, None]
    norm_ref[...] = norm
    h_ref[...] = feat_ref[...] * norm


_h_call = pl.pallas_call(
    _h_body,
    out_shape=(jax.ShapeDtypeStruct((N_NODES, FEATS), jnp.float32),
               jax.ShapeDtypeStruct((N_NODES, 1), jnp.float32)),
)


def _out_body(aggp_ref, w_ref, norm_ref, bias_ref, out_ref):
    agg = aggp_ref[0] + aggp_ref[1]
    r = jnp.dot(agg, w_ref[...], preferred_element_type=jnp.float32)
    out_ref[...] = r * norm_ref[...] + bias_ref[...]


_out_call = pl.pallas_call(
    _out_body,
    grid=(NRB,),
    in_specs=[
        pl.BlockSpec((NC, RB, FEATS), lambda i: (0, i, 0)),
        pl.BlockSpec((FEATS, FEATS), lambda i: (0, 0)),
        pl.BlockSpec((RB, 1), lambda i: (i, 0)),
        pl.BlockSpec((1, FEATS), lambda i: (0, 0)),
    ],
    out_specs=pl.BlockSpec((RB, FEATS), lambda i: (i, 0)),
    out_shape=jax.ShapeDtypeStruct((N_NODES, FEATS), jnp.float32),
)


def kernel(feat, edge_index, weight, bias):
    ei = edge_index.reshape(2, NC, NS, NG, G, CHUNK)
    degp = _deg_kernel(ei)
    h, norm = _h_call(degp, feat)
    aggp = _agg_kernel(h, ei)
    return _out_call(aggp, weight, norm, bias.reshape(1, FEATS))
